# baseline (speedup 1.0000x reference)
import jax
import jax.numpy as jnp
from jax.experimental import pallas as pl
from jax.experimental.pallas import tpu as pltpu
from jax.experimental.pallas import tpu_sc as plsc

_WINDOW = 128


def kernel(token_ids, weight):
    batch, seq = token_ids.shape
    num_rows, dim = weight.shape
    flat = batch * seq
    idx = token_ids.reshape(1, flat).astype(jnp.int32)

    mesh = plsc.VectorSubcoreMesh(core_axis_name="c", subcore_axis_name="s")

    @pl.kernel(
        out_type=jax.ShapeDtypeStruct((flat, dim), weight.dtype),
        mesh=mesh,
        compiler_params=pltpu.CompilerParams(use_tc_tiling_on_sc=False),
    )
    def gather_kernel(w_hbm, i_hbm, o_hbm):
        def body(i_vmem, o_vmem):
            pltpu.sync_copy(w_hbm.at[i_vmem.at[0]], o_vmem)

        pltpu.emit_pipeline(
            body,
            grid=(flat // _WINDOW,),
            in_specs=[pl.BlockSpec((1, _WINDOW), lambda i: (0, i))],
            out_specs=[pl.BlockSpec((_WINDOW, dim), lambda i: (i, 0))],
            core_axis_name=("c", "s"),
            dimension_semantics=(pltpu.PARALLEL,),
        )(i_hbm, o_hbm)

    out = gather_kernel(weight, idx)
    return out.reshape(batch, seq, dim)


# speedup vs baseline: 1.0690x; 1.0690x over previous
import jax
import jax.numpy as jnp
from jax.experimental import pallas as pl
from jax.experimental.pallas import tpu as pltpu
from jax.experimental.pallas import tpu_sc as plsc

_W = 128
_NBUF = 8
_NW = 32


def kernel(token_ids, weight):
    batch, seq = token_ids.shape
    num_rows, dim = weight.shape
    flat = batch * seq
    chunks_per_w = flat // (_NW * _W)
    idx = token_ids.reshape(_NW, chunks_per_w, _W).astype(jnp.int32)

    mesh = plsc.VectorSubcoreMesh(core_axis_name="c", subcore_axis_name="s")

    @pl.kernel(
        out_type=jax.ShapeDtypeStruct((flat, dim), weight.dtype),
        mesh=mesh,
        compiler_params=pltpu.CompilerParams(use_tc_tiling_on_sc=False),
        scratch_types=[
            pltpu.VMEM((chunks_per_w, _W), jnp.int32),
            pltpu.VMEM((_NBUF, _W, dim), jnp.float32),
            pltpu.SemaphoreType.DMA((_NBUF,)),
            pltpu.SemaphoreType.DMA((_NBUF,)),
            pltpu.SemaphoreType.DMA,
        ],
    )
    def gather_kernel(w_hbm, i_hbm, o_hbm, idx_v, rows_v, gsem, wsem, isem):
        wid = jax.lax.axis_index("s") * 2 + jax.lax.axis_index("c")
        base = wid * (chunks_per_w * _W)
        pltpu.async_copy(i_hbm.at[wid], idx_v, isem).wait()

        def start_gather(c, b):
            pltpu.async_copy(w_hbm.at[idx_v.at[c]], rows_v.at[b], gsem.at[b])

        def start_write(c, b):
            pltpu.async_copy(
                rows_v.at[b], o_hbm.at[pl.ds(base + c * _W, _W)], wsem.at[b]
            )

        for b in range(_NBUF):
            start_gather(b, b)

        @pl.loop(0, chunks_per_w, step=_NBUF)
        def _(g0):
            for b in range(_NBUF):
                c = g0 + b
                pltpu.make_async_copy(
                    w_hbm.at[idx_v.at[c]], rows_v.at[b], gsem.at[b]
                ).wait()
                start_write(c, b)
                pltpu.make_async_copy(
                    rows_v.at[b], o_hbm.at[pl.ds(base + c * _W, _W)], wsem.at[b]
                ).wait()

                @pl.when(c + _NBUF < chunks_per_w)
                def _():
                    start_gather(c + _NBUF, b)

    out = gather_kernel(weight, idx)
    return out.reshape(batch, seq, dim)
